# baseline (speedup 1.0000x reference)
import functools

import jax
import jax.numpy as jnp
from jax import lax
from jax.experimental import pallas as pl
from jax.experimental.pallas import tpu as pltpu
from jax.experimental.pallas import tpu_sc as plsc

B = 16384
G = 20
D = 64
NC = 2
NS = 16
NW = NC * NS
BPW = B // NW
WC = 32
NCHUNK = BPW // WC
IPC = WC * G
IB = 128
NGATH = IPC // IB
LG = D // 16


def _sc_body(crid_hbm, coff_hbm, nrid_hbm, noff_hbm, win2_hbm, wsub2_hbm,
             out2_hbm, cidx_v, nidx_v, coff_v, noff_v, crow_v, srow_v,
             orow_v, sem):
    wid = lax.axis_index("s") * NC + lax.axis_index("c")
    base = wid * BPW
    lanes = lax.iota(jnp.int32, 16)

    def chunk_body(ci, carry):
        wbase = base + ci * WC
        pltpu.sync_copy(crid_hbm.at[pl.ds(wbase, WC)], cidx_v)
        pltpu.sync_copy(nrid_hbm.at[pl.ds(wbase * G, IPC)], nidx_v)
        pltpu.sync_copy(coff_hbm.at[pl.ds(wbase, WC)], coff_v)
        pltpu.sync_copy(noff_hbm.at[pl.ds(wbase * G, IPC)], noff_v)

        cps = [pltpu.async_copy(win2_hbm.at[cidx_v], crow_v, sem)]
        for j in range(NGATH):
            cps.append(pltpu.async_copy(
                wsub2_hbm.at[nidx_v.at[pl.ds(j * IB, IB)]],
                srow_v.at[pl.ds(j * IB, IB)], sem))
        for cp in cps:
            cp.wait()

        def word_body(w, c):
            w16 = jnp.full((16,), w, jnp.int32)
            co = plsc.load_gather(coff_v, [w16])
            accs = [plsc.load_gather(crow_v, [w16, co + (16 * l + lanes)])
                    for l in range(LG)]
            for g in range(G):
                r = w * G + g
                r16 = jnp.full((16,), r, jnp.int32)
                no = plsc.load_gather(noff_v, [r16])
                for l in range(LG):
                    accs[l] = accs[l] + plsc.load_gather(
                        srow_v, [r16, no + (16 * l + lanes)])
            o16 = jnp.full((16,), w >> 1, jnp.int32)
            oo = jnp.full((16,), (w & 1) * 64, jnp.int32)
            for l in range(LG):
                plsc.store_scatter(orow_v, [o16, oo + (16 * l + lanes)],
                                   accs[l])
            return c

        lax.fori_loop(0, WC, word_body, 0)
        obase = pl.multiple_of(wbase // 2, WC // 2)
        pltpu.sync_copy(orow_v, out2_hbm.at[pl.ds(obase, WC // 2)])
        return carry

    lax.fori_loop(0, NCHUNK, chunk_body, 0)


_mesh = plsc.VectorSubcoreMesh(core_axis_name="c", subcore_axis_name="s")

_sc_call = functools.partial(
    pl.kernel,
    mesh=_mesh,
    out_type=jax.ShapeDtypeStruct((B // 2, 128), jnp.float32),
    scratch_types=[
        pltpu.VMEM((WC,), jnp.int32),
        pltpu.VMEM((IPC,), jnp.int32),
        pltpu.VMEM((WC,), jnp.int32),
        pltpu.VMEM((IPC,), jnp.int32),
        pltpu.VMEM((WC, 128), jnp.float32),
        pltpu.VMEM((IPC, 128), jnp.float32),
        pltpu.VMEM((WC // 2, 128), jnp.float32),
        pltpu.SemaphoreType.DMA,
    ],
    compiler_params=pltpu.CompilerParams(needs_layout_passes=False),
)(_sc_body)


def kernel(center_ids, ngram_ids, W_in, W_sub):
    cent = center_ids.astype(jnp.int32)
    ngr = ngram_ids.astype(jnp.int32).reshape(B * G)
    crid = cent >> 1
    coff = (cent & 1) * 64
    nrid = ngr >> 1
    noff = (ngr & 1) * 64
    win2 = W_in.reshape(W_in.shape[0] // 2, 128)
    wsub2 = W_sub.reshape(W_sub.shape[0] // 2, 128)
    out2 = _sc_call(crid, coff, nrid, noff, win2, wsub2)
    return out2.reshape(B, D)
